# baseline (speedup 1.0000x reference)
import functools

import jax
import jax.numpy as jnp
from jax import lax
from jax.experimental import pallas as pl
from jax.experimental.pallas import tpu as pltpu
from jax.experimental.pallas import tpu_sc as plsc

N = 10000
E = 160000
D_IN = 128
D_E = 16
H = 32
EH = 128
D_OUT = 64
STEPS = 6

W = 128
NPAD = 10240
EPAD = 163840
NC = 2
NS = 16
NW = NC * NS
EPW = EPAD // NW
CH = 128
NCH = EPW // CH
G = 2
GE = G * CH
NG = NCH // G
NSCG = NCH
ROWS_PER_SUB = NPAD // NS


def _proj_body(nf_ref, w1_ref, b1_ref, w2_ref, b2_ref, out_ref):
    h = jnp.maximum(jnp.dot(nf_ref[...], w1_ref[...],
                            preferred_element_type=jnp.float32) + b1_ref[...], 0.0)
    x = jnp.dot(h, w2_ref[...], preferred_element_type=jnp.float32) + b2_ref[...]
    out_ref[...] = jnp.concatenate(
        [x, jnp.zeros((x.shape[0], W - H), jnp.float32)], axis=1)


def _edgenet_body(ef_ref, w1_ref, b1_ref, w2_ref, b2_ref, out_ref):
    h = jnp.maximum(jnp.dot(ef_ref[...], w1_ref[...],
                            preferred_element_type=jnp.float32) + b1_ref[...], 0.0)
    ew = jax.lax.dot_general(h.astype(jnp.bfloat16), w2_ref[...],
                             (((1,), (0,)), ((), ())),
                             preferred_element_type=jnp.float32) + b2_ref[...]
    out_ref[...] = ew.astype(jnp.bfloat16)


def _mul_body(ew_ref, z_ref, sel_ref, out_ref):
    ew = ew_ref[...].astype(jnp.float32)
    zr = jnp.dot(z_ref[:, 0:H], sel_ref[...], preferred_element_type=jnp.float32)
    m = ew * zr
    w = H * H
    while w > H:
        w //= 2
        m = m[:, :w] + m[:, w:2 * w]
    out_ref[...] = jnp.concatenate(
        [m, jnp.zeros((m.shape[0], W - H), jnp.float32)], axis=1)


def _gru_body(agg_ref, hid_ref, wih_ref, bih_ref, whh_ref, bhh_ref, bconv_ref,
              out_ref):
    agg = agg_ref[0, :, 0:H] + agg_ref[1, :, 0:H] + bconv_ref[...]
    x = jnp.maximum(agg, 0.0)
    hid = hid_ref[:, 0:H]
    gi = jnp.dot(x, wih_ref[...], preferred_element_type=jnp.float32) + bih_ref[...]
    gh = jnp.dot(hid, whh_ref[...], preferred_element_type=jnp.float32) + bhh_ref[...]
    r = jax.nn.sigmoid(gi[:, 0:H] + gh[:, 0:H])
    zg = jax.nn.sigmoid(gi[:, H:2 * H] + gh[:, H:2 * H])
    n = jnp.tanh(gi[:, 2 * H:3 * H] + r * gh[:, 2 * H:3 * H])
    h_new = (1.0 - zg) * n + zg * hid
    out_ref[...] = jnp.concatenate(
        [h_new, jnp.zeros((h_new.shape[0], W - H), jnp.float32)], axis=1)


def _dec_body(x_ref, w1_ref, b1_ref, w2_ref, b2_ref, out_ref):
    h = jnp.maximum(jnp.dot(x_ref[:, 0:H], w1_ref[...],
                            preferred_element_type=jnp.float32) + b1_ref[...], 0.0)
    out_ref[...] = jnp.dot(h, w2_ref[...],
                           preferred_element_type=jnp.float32) + b2_ref[...]


def _full(shape):
    return pl.BlockSpec(shape, lambda *_: tuple(0 for _ in shape))


def _proj(nf, w1, b1, w2, b2):
    return pl.pallas_call(
        _proj_body,
        grid=(1,),
        in_specs=[_full((NPAD, D_IN)), _full((D_IN, H)), _full((1, H)),
                  _full((H, H)), _full((1, H))],
        out_specs=_full((NPAD, W)),
        out_shape=jax.ShapeDtypeStruct((NPAD, W), jnp.float32),
    )(nf, w1, b1, w2, b2)


_ET = 512


def _edgenet(ef, w1, b1, w2, b2):
    return pl.pallas_call(
        _edgenet_body,
        grid=(EPAD // _ET,),
        in_specs=[pl.BlockSpec((_ET, D_E), lambda i: (i, 0)),
                  _full((D_E, EH)), _full((1, EH)),
                  _full((EH, H * H)), _full((1, H * H))],
        out_specs=pl.BlockSpec((_ET, H * H), lambda i: (i, 0)),
        out_shape=jax.ShapeDtypeStruct((EPAD, H * H), jnp.bfloat16),
    )(ef, w1, b1, w2, b2)


def _mul(ew, z, sel):
    return pl.pallas_call(
        _mul_body,
        grid=(EPAD // _ET,),
        in_specs=[pl.BlockSpec((_ET, H * H), lambda i: (i, 0)),
                  pl.BlockSpec((_ET, W), lambda i: (i, 0)),
                  _full((H, H * H))],
        out_specs=pl.BlockSpec((_ET, W), lambda i: (i, 0)),
        out_shape=jax.ShapeDtypeStruct((EPAD, W), jnp.float32),
    )(ew, z, sel)


def _gru(aggpair, hid, wih, bih, whh, bhh, bconv):
    return pl.pallas_call(
        _gru_body,
        grid=(1,),
        in_specs=[_full((2, NPAD, W)), _full((NPAD, W)),
                  _full((H, 3 * H)), _full((1, 3 * H)),
                  _full((H, 3 * H)), _full((1, 3 * H)), _full((1, H))],
        out_specs=_full((NPAD, W)),
        out_shape=jax.ShapeDtypeStruct((NPAD, W), jnp.float32),
    )(aggpair, hid, wih, bih, whh, bhh, bconv)


def _dec(x, w1, b1, w2, b2):
    return pl.pallas_call(
        _dec_body,
        grid=(1,),
        in_specs=[_full((NPAD, W)), _full((H, H)), _full((1, H)),
                  _full((H, D_OUT)), _full((1, D_OUT))],
        out_specs=_full((NPAD, D_OUT)),
        out_shape=jax.ShapeDtypeStruct((NPAD, D_OUT), jnp.float32),
    )(x, w1, b1, w2, b2)


def _sc_gather_body(x_hbm, src_hbm, out_hbm, idx_v, buf, sem_g, sem_w):
    c = lax.axis_index("c")
    s = lax.axis_index("s")
    w = c * NS + s
    base = w * EPW
    pltpu.sync_copy(src_hbm.at[pl.ds(w * NCH, NCH)], idx_v)

    def fire_gather(g, b):
        for q in range(G):
            pltpu.async_copy(x_hbm.at[idx_v.at[g * G + q]],
                             buf.at[b, pl.ds(q * CH, CH)], sem_g)

    def wait_gather(b):
        pltpu.make_async_copy(x_hbm.at[pl.ds(0, GE)], buf.at[b], sem_g).wait()

    def fire_write(g, b):
        pltpu.async_copy(buf.at[b], out_hbm.at[pl.ds(base + g * GE, GE)], sem_w)

    def wait_one_write(b):
        pltpu.make_async_copy(buf.at[b],
                              out_hbm.at[pl.ds(base, GE)], sem_w).wait()

    fire_gather(0, 0)
    fire_gather(1, 1)

    def pair(p, carry):
        g0 = p * 2
        wait_gather(0)
        fire_write(g0, 0)
        wait_one_write(0)
        fire_gather(g0 + 2, 0)
        wait_gather(1)
        fire_write(g0 + 1, 1)
        wait_one_write(1)
        fire_gather(g0 + 3, 1)
        return carry

    lax.fori_loop(0, NG // 2 - 1, pair, 0)
    wait_gather(0)
    fire_write(NG - 2, 0)
    wait_gather(1)
    fire_write(NG - 1, 1)
    wait_one_write(0)
    wait_one_write(1)


def _sc_scatter_body(msg_hbm, dst_hbm, zeros_hbm, out_hbm, idx_v, buf, acc_sh,
                     sem_l, sem_a):
    c = lax.axis_index("c")
    s = lax.axis_index("s")
    w = c * NS + s
    base = w * EPW

    def fire_load(g, b):
        pltpu.async_copy(msg_hbm.at[pl.ds(base + g * CH, CH)], buf.at[b], sem_l)

    def wait_load(b):
        pltpu.make_async_copy(msg_hbm.at[pl.ds(base, CH)], buf.at[b],
                              sem_l).wait()

    def fire_add(g, b):
        pltpu.async_copy(buf.at[b], acc_sh.at[idx_v.at[g]], sem_a, add=True)

    def wait_one_add(b):
        pltpu.make_async_copy(buf.at[b], acc_sh.at[pl.ds(0, CH)], sem_a).wait()

    fire_load(0, 0)
    fire_load(1, 1)
    pltpu.sync_copy(zeros_hbm.at[pl.ds(s * ROWS_PER_SUB, ROWS_PER_SUB)],
                    acc_sh.at[pl.ds(s * ROWS_PER_SUB, ROWS_PER_SUB)])
    pltpu.sync_copy(dst_hbm.at[pl.ds(w * NCH, NCH)], idx_v)
    plsc.subcore_barrier()

    def pair(p, carry):
        g0 = p * 2
        wait_load(0)
        fire_add(g0, 0)
        wait_load(1)
        fire_add(g0 + 1, 1)
        wait_one_add(0)
        fire_load(g0 + 2, 0)
        wait_one_add(1)
        fire_load(g0 + 3, 1)
        return carry

    lax.fori_loop(0, NSCG // 2 - 1, pair, 0)
    wait_load(0)
    fire_add(NSCG - 2, 0)
    wait_load(1)
    fire_add(NSCG - 1, 1)
    wait_one_add(0)
    wait_one_add(1)
    plsc.subcore_barrier()
    pltpu.sync_copy(acc_sh.at[pl.ds(s * ROWS_PER_SUB, ROWS_PER_SUB)],
                    out_hbm.at[c].at[pl.ds(s * ROWS_PER_SUB, ROWS_PER_SUB)])


@functools.cache
def _sc_kernels():
    mesh = plsc.VectorSubcoreMesh(core_axis_name="c", subcore_axis_name="s",
                                  num_cores=NC, num_subcores=NS)
    gather = pl.kernel(
        _sc_gather_body,
        out_type=jax.ShapeDtypeStruct((EPAD, W), jnp.float32),
        mesh=mesh,
        scratch_types=[pltpu.VMEM((NCH, CH), jnp.int32),
                       pltpu.VMEM((2, GE, W), jnp.float32),
                       pltpu.SemaphoreType.DMA,
                       pltpu.SemaphoreType.DMA],
    )
    scatter = pl.kernel(
        _sc_scatter_body,
        out_type=jax.ShapeDtypeStruct((NC, NPAD, W), jnp.float32),
        mesh=mesh,
        scratch_types=[pltpu.VMEM((NCH, CH), jnp.int32),
                       pltpu.VMEM((2, CH, W), jnp.float32),
                       pltpu.VMEM_SHARED((NPAD, W), jnp.float32),
                       pltpu.SemaphoreType.DMA,
                       pltpu.SemaphoreType.DMA],
    )
    return gather, scatter


def kernel(node_feats, edge_feats, edge_index, W_p1, b_p1, W_p2, b_p2,
           W_e1, b_e1, W_e2, b_e2, b_conv, W_ih, b_ih, W_hh, b_hh,
           W_d1, b_d1, W_d2, b_d2):
    nf = jnp.pad(node_feats, ((0, NPAD - N), (0, 0)))
    ef = jnp.pad(edge_feats, ((0, EPAD - E), (0, 0)))
    src = jnp.pad(edge_index[0].astype(jnp.int32), (0, EPAD - E))
    dst = jnp.pad(edge_index[1].astype(jnp.int32), (0, EPAD - E),
                  constant_values=N)
    src2d = src.reshape(EPAD // CH, CH)
    dst2d = dst.reshape(EPAD // CH, CH)
    zeros_acc = jnp.zeros((NPAD, W), jnp.float32)
    lane = jnp.arange(H * H, dtype=jnp.int32) // H
    sel = (jnp.arange(H, dtype=jnp.int32)[:, None] == lane[None, :]
           ).astype(jnp.float32)

    x0 = _proj(nf, W_p1, b_p1.reshape(1, H), W_p2, b_p2.reshape(1, H))
    ew = _edgenet(ef, W_e1, b_e1.reshape(1, EH), W_e2.astype(jnp.bfloat16),
                  b_e2.reshape(1, H * H))

    wih = W_ih.T
    whh = W_hh.T
    bih = b_ih.reshape(1, 3 * H)
    bhh = b_hh.reshape(1, 3 * H)
    bconv = b_conv.reshape(1, H)

    sc_gather, sc_scatter = _sc_kernels()
    x = x0
    hidden = x0
    for _ in range(STEPS):
        z = sc_gather(x, src2d)
        msg = _mul(ew, z, sel)
        aggpair = sc_scatter(msg, dst2d, zeros_acc)
        hidden = _gru(aggpair, hidden, wih, bih, whh, bhh, bconv)
        x = hidden

    out = _dec(x, W_d1, b_d1.reshape(1, H), W_d2, b_d2.reshape(1, D_OUT))
    return out[:N]


# speedup vs baseline: 1.0245x; 1.0245x over previous
import functools

import jax
import jax.numpy as jnp
from jax import lax
from jax.experimental import pallas as pl
from jax.experimental.pallas import tpu as pltpu
from jax.experimental.pallas import tpu_sc as plsc

N = 10000
E = 160000
D_IN = 128
D_E = 16
H = 32
EH = 128
D_OUT = 64
STEPS = 6

W = 128
NPAD = 10240
EPAD = 163840
NC = 2
NS = 16
NW = NC * NS
EPW = EPAD // NW
CH = 128
NCH = EPW // CH
G = 2
GE = G * CH
NG = NCH // G
NSCG = NCH
EPADH = EPAD // 2
EPWH = EPADH // NW
NCHH = EPWH // CH
NGH = NCHH // G
ROWS_PER_SUB = NPAD // NS
ROWS4_PER_SUB = NPAD // 4 // NS


def _proj_body(nf_ref, w1_ref, b1_ref, w2_ref, b2_ref, out_ref):
    h = jnp.maximum(jnp.dot(nf_ref[...], w1_ref[...],
                            preferred_element_type=jnp.float32) + b1_ref[...], 0.0)
    x = jnp.dot(h, w2_ref[...], preferred_element_type=jnp.float32) + b2_ref[...]
    out_ref[...] = jnp.concatenate(
        [x, jnp.zeros((x.shape[0], W - H), jnp.float32)], axis=1)


def _edgenet_body(ef_ref, w1_ref, b1_ref, w2_ref, b2_ref, out_ref):
    h = jnp.maximum(jnp.dot(ef_ref[...], w1_ref[...],
                            preferred_element_type=jnp.float32) + b1_ref[...], 0.0)
    ew = jax.lax.dot_general(h.astype(jnp.bfloat16), w2_ref[...],
                             (((1,), (0,)), ((), ())),
                             preferred_element_type=jnp.float32) + b2_ref[...]
    out_ref[...] = ew.astype(jnp.bfloat16)


def _mul_body(ew_ref, z_ref, sel_ref, out_ref):
    ew = ew_ref[...].astype(jnp.float32)
    zr = jnp.dot(z_ref[:, 0:H], sel_ref[...], preferred_element_type=jnp.float32)
    m = ew * zr
    w = H * H
    while w > H:
        w //= 2
        m = m[:, :w] + m[:, w:2 * w]
    out_ref[...] = jnp.concatenate(
        [m, jnp.zeros((m.shape[0], W - H), jnp.float32)], axis=1)


def _gru_body(agga_ref, aggb_ref, hid_ref, wih_ref, bih_ref, whh_ref,
              bhh_ref, bconv_ref, out_ref):
    agg = (agga_ref[0, :, 0:H] + agga_ref[1, :, 0:H]
           + aggb_ref[0, :, 0:H] + aggb_ref[1, :, 0:H] + bconv_ref[...])
    x = jnp.maximum(agg, 0.0)
    hid = hid_ref[:, 0:H]
    gi = jnp.dot(x, wih_ref[...], preferred_element_type=jnp.float32) + bih_ref[...]
    gh = jnp.dot(hid, whh_ref[...], preferred_element_type=jnp.float32) + bhh_ref[...]
    r = jax.nn.sigmoid(gi[:, 0:H] + gh[:, 0:H])
    zg = jax.nn.sigmoid(gi[:, H:2 * H] + gh[:, H:2 * H])
    n = jnp.tanh(gi[:, 2 * H:3 * H] + r * gh[:, 2 * H:3 * H])
    h_new = (1.0 - zg) * n + zg * hid
    out_ref[...] = jnp.concatenate(
        [h_new, jnp.zeros((h_new.shape[0], W - H), jnp.float32)], axis=1)


def _dec_body(x_ref, w1_ref, b1_ref, w2_ref, b2_ref, out_ref):
    h = jnp.maximum(jnp.dot(x_ref[:, 0:H], w1_ref[...],
                            preferred_element_type=jnp.float32) + b1_ref[...], 0.0)
    out_ref[...] = jnp.dot(h, w2_ref[...],
                           preferred_element_type=jnp.float32) + b2_ref[...]


def _full(shape):
    return pl.BlockSpec(shape, lambda *_: tuple(0 for _ in shape))


def _proj(nf, w1, b1, w2, b2):
    return pl.pallas_call(
        _proj_body,
        grid=(1,),
        in_specs=[_full((NPAD, D_IN)), _full((D_IN, H)), _full((1, H)),
                  _full((H, H)), _full((1, H))],
        out_specs=_full((NPAD, W)),
        out_shape=jax.ShapeDtypeStruct((NPAD, W), jnp.float32),
    )(nf, w1, b1, w2, b2)


_ET = 512


def _edgenet(ef, w1, b1, w2, b2):
    return pl.pallas_call(
        _edgenet_body,
        grid=(EPAD // _ET,),
        in_specs=[pl.BlockSpec((_ET, D_E), lambda i: (i, 0)),
                  _full((D_E, EH)), _full((1, EH)),
                  _full((EH, H * H)), _full((1, H * H))],
        out_specs=pl.BlockSpec((_ET, H * H), lambda i: (i, 0)),
        out_shape=jax.ShapeDtypeStruct((EPAD, H * H), jnp.bfloat16),
    )(ef, w1, b1, w2, b2)


def _mul(ew, z, sel, half):
    toff = half * (EPADH // _ET)
    return pl.pallas_call(
        _mul_body,
        grid=(EPADH // _ET,),
        in_specs=[pl.BlockSpec((_ET, H * H), lambda i, t=toff: (i + t, 0)),
                  pl.BlockSpec((_ET, W), lambda i: (i, 0)),
                  _full((H, H * H))],
        out_specs=pl.BlockSpec((_ET, W), lambda i: (i, 0)),
        out_shape=jax.ShapeDtypeStruct((EPADH, W), jnp.float32),
    )(ew, z, sel)


def _gru(agga, aggb, hid, wih, bih, whh, bhh, bconv):
    return pl.pallas_call(
        _gru_body,
        grid=(1,),
        in_specs=[_full((2, NPAD, W)), _full((2, NPAD, W)), _full((NPAD, W)),
                  _full((H, 3 * H)), _full((1, 3 * H)),
                  _full((H, 3 * H)), _full((1, 3 * H)), _full((1, H))],
        out_specs=_full((NPAD, W)),
        out_shape=jax.ShapeDtypeStruct((NPAD, W), jnp.float32),
    )(agga, aggb, hid, wih, bih, whh, bhh, bconv)


def _dec(x, w1, b1, w2, b2):
    return pl.pallas_call(
        _dec_body,
        grid=(1,),
        in_specs=[_full((NPAD, W)), _full((H, H)), _full((1, H)),
                  _full((H, D_OUT)), _full((1, D_OUT))],
        out_specs=_full((NPAD, D_OUT)),
        out_shape=jax.ShapeDtypeStruct((NPAD, D_OUT), jnp.float32),
    )(x, w1, b1, w2, b2)


def _sc_gather_body(x_hbm, src_hbm, out_hbm, idx_v, buf, sem_g, sem_w):
    c = lax.axis_index("c")
    s = lax.axis_index("s")
    w = c * NS + s
    base = w * EPWH
    pltpu.sync_copy(src_hbm.at[w], idx_v)

    def fire_gather(g, b):
        for q in range(G):
            pltpu.async_copy(x_hbm.at[idx_v.at[g * G + q]],
                             buf.at[b, pl.ds(q * CH, CH)], sem_g)

    def wait_gather(b):
        pltpu.make_async_copy(x_hbm.at[pl.ds(0, GE)], buf.at[b], sem_g).wait()

    def fire_write(g, b):
        pltpu.async_copy(buf.at[b], out_hbm.at[pl.ds(base + g * GE, GE)], sem_w)

    def wait_one_write(b):
        pltpu.make_async_copy(buf.at[b],
                              out_hbm.at[pl.ds(base, GE)], sem_w).wait()

    fire_gather(0, 0)
    fire_gather(1, 1)

    def pair(p, carry):
        g0 = p * 2
        wait_gather(0)
        fire_write(g0, 0)
        wait_one_write(0)
        fire_gather(g0 + 2, 0)
        wait_gather(1)
        fire_write(g0 + 1, 1)
        wait_one_write(1)
        fire_gather(g0 + 3, 1)
        return carry

    lax.fori_loop(0, NGH // 2 - 1, pair, 0)
    wait_gather(0)
    fire_write(NGH - 2, 0)
    wait_gather(1)
    fire_write(NGH - 1, 1)
    wait_one_write(0)
    wait_one_write(1)


def _sc_scatter_body(msg_hbm, dst_hbm, zeros_hbm, out_hbm, idx_v, buf, acc_sh,
                     sem_l, sem_a):
    c = lax.axis_index("c")
    s = lax.axis_index("s")
    w = c * NS + s
    base = w * EPWH

    def fire_load(g, b):
        pltpu.async_copy(msg_hbm.at[pl.ds(base + g * CH, CH)], buf.at[b], sem_l)

    def wait_load(b):
        pltpu.make_async_copy(msg_hbm.at[pl.ds(base, CH)], buf.at[b],
                              sem_l).wait()

    def fire_add(g, b):
        pltpu.async_copy(buf.at[b], acc_sh.at[idx_v.at[g]], sem_a, add=True)

    def wait_one_add(b):
        pltpu.make_async_copy(buf.at[b], acc_sh.at[pl.ds(0, CH)], sem_a).wait()

    fire_load(0, 0)
    fire_load(1, 1)
    pltpu.sync_copy(zeros_hbm.at[pl.ds(s * ROWS_PER_SUB, ROWS_PER_SUB)],
                    acc_sh.at[pl.ds(s * ROWS_PER_SUB, ROWS_PER_SUB)])
    pltpu.sync_copy(dst_hbm.at[w], idx_v)
    plsc.subcore_barrier()

    def pair(p, carry):
        g0 = p * 2
        wait_load(0)
        fire_add(g0, 0)
        wait_load(1)
        fire_add(g0 + 1, 1)
        wait_one_add(0)
        fire_load(g0 + 2, 0)
        wait_one_add(1)
        fire_load(g0 + 3, 1)
        return carry

    lax.fori_loop(0, NCHH // 2 - 1, pair, 0)
    wait_load(0)
    fire_add(NCHH - 2, 0)
    wait_load(1)
    fire_add(NCHH - 1, 1)
    wait_one_add(0)
    wait_one_add(1)
    plsc.subcore_barrier()
    pltpu.sync_copy(acc_sh.at[pl.ds(s * ROWS_PER_SUB, ROWS_PER_SUB)],
                    out_hbm.at[c].at[pl.ds(s * ROWS_PER_SUB, ROWS_PER_SUB)])


@functools.cache
def _sc_kernels():
    mesh = plsc.VectorSubcoreMesh(core_axis_name="c", subcore_axis_name="s",
                                  num_cores=NC, num_subcores=NS)
    gather = pl.kernel(
        _sc_gather_body,
        out_type=jax.ShapeDtypeStruct((EPADH, W), jnp.float32),
        mesh=mesh,
        scratch_types=[pltpu.VMEM((NCHH, CH), jnp.int32),
                       pltpu.VMEM((2, GE, W), jnp.float32),
                       pltpu.SemaphoreType.DMA,
                       pltpu.SemaphoreType.DMA],
    )
    scatter = pl.kernel(
        _sc_scatter_body,
        out_type=jax.ShapeDtypeStruct((NC, NPAD, W), jnp.float32),
        mesh=mesh,
        scratch_types=[pltpu.VMEM((NCHH, CH), jnp.int32),
                       pltpu.VMEM((2, CH, W), jnp.float32),
                       pltpu.VMEM_SHARED((NPAD, W), jnp.float32),
                       pltpu.SemaphoreType.DMA,
                       pltpu.SemaphoreType.DMA],
    )
    return gather, scatter


def kernel(node_feats, edge_feats, edge_index, W_p1, b_p1, W_p2, b_p2,
           W_e1, b_e1, W_e2, b_e2, b_conv, W_ih, b_ih, W_hh, b_hh,
           W_d1, b_d1, W_d2, b_d2):
    nf = jnp.pad(node_feats, ((0, NPAD - N), (0, 0)))
    ef = jnp.pad(edge_feats, ((0, EPAD - E), (0, 0)))
    src = jnp.pad(edge_index[0].astype(jnp.int32), (0, EPAD - E))
    dst = jnp.pad(edge_index[1].astype(jnp.int32), (0, EPAD - E),
                  constant_values=N)
    src2d = src.reshape(EPAD // CH, CH)
    dst2d = dst.reshape(EPAD // CH, CH)
    zeros_acc = jnp.zeros((NPAD, W), jnp.float32)
    lane = jnp.arange(H * H, dtype=jnp.int32) // H
    sel = (jnp.arange(H, dtype=jnp.int32)[:, None] == lane[None, :]
           ).astype(jnp.float32)

    x0 = _proj(nf, W_p1, b_p1.reshape(1, H), W_p2, b_p2.reshape(1, H))
    ew = _edgenet(ef, W_e1, b_e1.reshape(1, EH), W_e2.astype(jnp.bfloat16),
                  b_e2.reshape(1, H * H))

    wih = W_ih.T
    whh = W_hh.T
    bih = b_ih.reshape(1, 3 * H)
    bhh = b_hh.reshape(1, 3 * H)
    bconv = b_conv.reshape(1, H)

    nrow_h = EPADH // CH
    srcA = src2d[:nrow_h].reshape(NW, NCHH, CH)
    srcB = src2d[nrow_h:].reshape(NW, NCHH, CH)
    dstA = dst2d[:nrow_h].reshape(NW, NCHH, CH)
    dstB = dst2d[nrow_h:].reshape(NW, NCHH, CH)

    sc_gather, sc_scatter = _sc_kernels()
    x = x0
    hidden = x0
    for _ in range(STEPS):
        zA = sc_gather(x, srcA)
        msgA = _mul(ew, zA, sel, 0)
        zB = sc_gather(x, srcB)
        msgB = _mul(ew, zB, sel, 1)
        aggA = sc_scatter(msgA, dstA, zeros_acc)
        aggB = sc_scatter(msgB, dstB, zeros_acc)
        hidden = _gru(aggA, aggB, hidden, wih, bih, whh, bhh, bconv)
        x = hidden

    out = _dec(x, W_d1, b_d1.reshape(1, H), W_d2, b_d2.reshape(1, D_OUT))
    return out[:N]


# speedup vs baseline: 1.0357x; 1.0109x over previous
import functools

import jax
import jax.numpy as jnp
from jax import lax
from jax.experimental import pallas as pl
from jax.experimental.pallas import tpu as pltpu
from jax.experimental.pallas import tpu_sc as plsc

N = 10000
E = 160000
D_IN = 128
D_E = 16
H = 32
EH = 128
D_OUT = 64
STEPS = 6

W = 128
NPAD = 10240
EPAD = 163840
NC = 2
NS = 16
NW = NC * NS
EPW = EPAD // NW
CH = 128
NCH = EPW // CH
G = 2
GE = G * CH
NG = NCH // G
NSCG = NCH
EPADH = EPAD // 2
EPWH = EPADH // NW
NCHH = EPWH // CH
NGH = NCHH // G
ROWS_PER_SUB = NPAD // NS
ROWS4_PER_SUB = NPAD // 4 // NS


def _proj_body(nf_ref, w1_ref, b1_ref, w2_ref, b2_ref, out_ref):
    h = jnp.maximum(jnp.dot(nf_ref[...], w1_ref[...],
                            preferred_element_type=jnp.float32) + b1_ref[...], 0.0)
    x = jnp.dot(h, w2_ref[...], preferred_element_type=jnp.float32) + b2_ref[...]
    out_ref[...] = jnp.concatenate(
        [x, jnp.zeros((x.shape[0], W - H), jnp.float32)], axis=1)


def _edgenet_body(ef_ref, w1_ref, b1_ref, w2_ref, b2_ref, out_ref):
    h = jnp.maximum(jnp.dot(ef_ref[...], w1_ref[...],
                            preferred_element_type=jnp.float32) + b1_ref[...], 0.0)
    ew = jax.lax.dot_general(h.astype(jnp.bfloat16), w2_ref[...],
                             (((1,), (0,)), ((), ())),
                             preferred_element_type=jnp.float32) + b2_ref[...]
    out_ref[...] = ew.astype(jnp.bfloat16)


def _mul_body(ew_ref, z_ref, sel_ref, out_ref):
    ew = ew_ref[...].astype(jnp.float32)
    zr = jnp.dot(z_ref[:, 0:H], sel_ref[...], preferred_element_type=jnp.float32)
    m = ew * zr
    w = H * H
    while w > H:
        w //= 2
        m = m[:, :w] + m[:, w:2 * w]
    out_ref[...] = jnp.concatenate(
        [m, jnp.zeros((m.shape[0], W - H), jnp.float32)], axis=1)


def _gru_body(agga_ref, aggb_ref, hid_ref, wih_ref, bih_ref, whh_ref,
              bhh_ref, bconv_ref, out_ref):
    agg = (agga_ref[0, :, 0:H] + agga_ref[1, :, 0:H]
           + aggb_ref[0, :, 0:H] + aggb_ref[1, :, 0:H] + bconv_ref[...])
    x = jnp.maximum(agg, 0.0)
    hid = hid_ref[:, 0:H]
    gi = jnp.dot(x, wih_ref[...], preferred_element_type=jnp.float32) + bih_ref[...]
    gh = jnp.dot(hid, whh_ref[...], preferred_element_type=jnp.float32) + bhh_ref[...]
    r = jax.nn.sigmoid(gi[:, 0:H] + gh[:, 0:H])
    zg = jax.nn.sigmoid(gi[:, H:2 * H] + gh[:, H:2 * H])
    n = jnp.tanh(gi[:, 2 * H:3 * H] + r * gh[:, 2 * H:3 * H])
    h_new = (1.0 - zg) * n + zg * hid
    out_ref[...] = jnp.concatenate(
        [h_new, jnp.zeros((h_new.shape[0], W - H), jnp.float32)], axis=1)


def _dec_body(x_ref, w1_ref, b1_ref, w2_ref, b2_ref, out_ref):
    h = jnp.maximum(jnp.dot(x_ref[:, 0:H], w1_ref[...],
                            preferred_element_type=jnp.float32) + b1_ref[...], 0.0)
    out_ref[...] = jnp.dot(h, w2_ref[...],
                           preferred_element_type=jnp.float32) + b2_ref[...]


def _full(shape):
    return pl.BlockSpec(shape, lambda *_: tuple(0 for _ in shape))


def _proj(nf, w1, b1, w2, b2):
    return pl.pallas_call(
        _proj_body,
        grid=(1,),
        in_specs=[_full((NPAD, D_IN)), _full((D_IN, H)), _full((1, H)),
                  _full((H, H)), _full((1, H))],
        out_specs=_full((NPAD, W)),
        out_shape=jax.ShapeDtypeStruct((NPAD, W), jnp.float32),
    )(nf, w1, b1, w2, b2)


_ET = 512


def _edgenet(ef, w1, b1, w2, b2):
    return pl.pallas_call(
        _edgenet_body,
        grid=(EPAD // _ET,),
        in_specs=[pl.BlockSpec((_ET, D_E), lambda i: (i, 0)),
                  _full((D_E, EH)), _full((1, EH)),
                  _full((EH, H * H)), _full((1, H * H))],
        out_specs=pl.BlockSpec((_ET, H * H), lambda i: (i, 0)),
        out_shape=jax.ShapeDtypeStruct((EPAD, H * H), jnp.bfloat16),
    )(ef, w1, b1, w2, b2)


def _mul(ew, z, sel, half):
    toff = half * (EPADH // _ET)
    return pl.pallas_call(
        _mul_body,
        grid=(EPADH // _ET,),
        in_specs=[pl.BlockSpec((_ET, H * H), lambda i, t=toff: (i + t, 0)),
                  pl.BlockSpec((_ET, W), lambda i: (i, 0)),
                  _full((H, H * H))],
        out_specs=pl.BlockSpec((_ET, W), lambda i: (i, 0)),
        out_shape=jax.ShapeDtypeStruct((EPADH, W), jnp.float32),
    )(ew, z, sel)


def _gru(agga, aggb, hid, wih, bih, whh, bhh, bconv):
    return pl.pallas_call(
        _gru_body,
        grid=(1,),
        in_specs=[_full((2, NPAD, W)), _full((2, NPAD, W)), _full((NPAD, W)),
                  _full((H, 3 * H)), _full((1, 3 * H)),
                  _full((H, 3 * H)), _full((1, 3 * H)), _full((1, H))],
        out_specs=_full((NPAD, W)),
        out_shape=jax.ShapeDtypeStruct((NPAD, W), jnp.float32),
    )(agga, aggb, hid, wih, bih, whh, bhh, bconv)


def _dec(x, w1, b1, w2, b2):
    return pl.pallas_call(
        _dec_body,
        grid=(1,),
        in_specs=[_full((NPAD, W)), _full((H, H)), _full((1, H)),
                  _full((H, D_OUT)), _full((1, D_OUT))],
        out_specs=_full((NPAD, D_OUT)),
        out_shape=jax.ShapeDtypeStruct((NPAD, D_OUT), jnp.float32),
    )(x, w1, b1, w2, b2)


def _sc_gather_body(x_hbm, src_hbm, out_hbm, idx_v, buf, sem_g, sem_w):
    c = lax.axis_index("c")
    s = lax.axis_index("s")
    w = c * NS + s
    base = w * EPWH
    pltpu.sync_copy(src_hbm.at[w], idx_v)

    def fire_gather(g, b):
        for q in range(G):
            pltpu.async_copy(x_hbm.at[idx_v.at[g * G + q]],
                             buf.at[b, pl.ds(q * CH, CH)], sem_g)

    def wait_gather(b):
        pltpu.make_async_copy(x_hbm.at[pl.ds(0, GE)], buf.at[b], sem_g).wait()

    def fire_write(g, b):
        pltpu.async_copy(buf.at[b], out_hbm.at[pl.ds(base + g * GE, GE)], sem_w)

    def wait_one_write(b):
        pltpu.make_async_copy(buf.at[b],
                              out_hbm.at[pl.ds(base, GE)], sem_w).wait()

    fire_gather(0, 0)
    fire_gather(1, 1)

    def pair(p, carry):
        g0 = p * 2
        wait_gather(0)
        fire_write(g0, 0)
        wait_one_write(0)
        fire_gather(g0 + 2, 0)
        wait_gather(1)
        fire_write(g0 + 1, 1)
        wait_one_write(1)
        fire_gather(g0 + 3, 1)
        return carry

    lax.fori_loop(0, NGH // 2 - 1, pair, 0)
    wait_gather(0)
    fire_write(NGH - 2, 0)
    wait_gather(1)
    fire_write(NGH - 1, 1)
    wait_one_write(0)
    wait_one_write(1)


def _sc_scatter_body(msg_hbm, dst_hbm, zeros_hbm, out_hbm, idx_v, buf, acc_sh,
                     sem_l, sem_a):
    c = lax.axis_index("c")
    s = lax.axis_index("s")
    w = c * NS + s
    base = w * EPWH

    def fire_load(g, b):
        pltpu.async_copy(msg_hbm.at[pl.ds(base + g * CH, CH)], buf.at[b], sem_l)

    def wait_load(b):
        pltpu.make_async_copy(msg_hbm.at[pl.ds(base, CH)], buf.at[b],
                              sem_l).wait()

    def fire_add(g, b):
        pltpu.async_copy(buf.at[b], acc_sh.at[idx_v.at[g]], sem_a, add=True)

    def wait_one_add(b):
        pltpu.make_async_copy(buf.at[b], acc_sh.at[pl.ds(0, CH)], sem_a).wait()

    fire_load(0, 0)
    fire_load(1, 1)
    pltpu.sync_copy(zeros_hbm.at[pl.ds(s * ROWS_PER_SUB, ROWS_PER_SUB)],
                    acc_sh.at[pl.ds(s * ROWS_PER_SUB, ROWS_PER_SUB)])
    pltpu.sync_copy(dst_hbm.at[w], idx_v)
    plsc.subcore_barrier()

    def pair(p, carry):
        g0 = p * 2
        wait_load(0)
        fire_add(g0, 0)
        wait_load(1)
        fire_add(g0 + 1, 1)
        wait_one_add(0)
        fire_load(g0 + 2, 0)
        wait_one_add(1)
        fire_load(g0 + 3, 1)
        return carry

    lax.fori_loop(0, NCHH // 2 - 1, pair, 0)
    wait_load(0)
    fire_add(NCHH - 2, 0)
    wait_load(1)
    fire_add(NCHH - 1, 1)
    wait_one_add(0)
    wait_one_add(1)
    plsc.subcore_barrier()
    pltpu.sync_copy(acc_sh.at[pl.ds(s * ROWS_PER_SUB, ROWS_PER_SUB)],
                    out_hbm.at[c].at[pl.ds(s * ROWS_PER_SUB, ROWS_PER_SUB)])


@functools.cache
def _sc_kernels():
    mesh = plsc.VectorSubcoreMesh(core_axis_name="c", subcore_axis_name="s",
                                  num_cores=NC, num_subcores=NS)
    gather = pl.kernel(
        _sc_gather_body,
        out_type=jax.ShapeDtypeStruct((EPADH, W), jnp.float32),
        mesh=mesh,
        scratch_types=[pltpu.VMEM((NCHH, CH), jnp.int32),
                       pltpu.VMEM((2, GE, W), jnp.float32),
                       pltpu.SemaphoreType.DMA,
                       pltpu.SemaphoreType.DMA],
    )
    scatter = pl.kernel(
        _sc_scatter_body,
        out_type=jax.ShapeDtypeStruct((NC, NPAD, W), jnp.float32),
        mesh=mesh,
        scratch_types=[pltpu.VMEM((NCHH, CH), jnp.int32),
                       pltpu.VMEM((2, CH, W), jnp.float32),
                       pltpu.VMEM_SHARED((NPAD, W), jnp.float32),
                       pltpu.SemaphoreType.DMA,
                       pltpu.SemaphoreType.DMA],
    )
    return gather, scatter


def kernel(node_feats, edge_feats, edge_index, W_p1, b_p1, W_p2, b_p2,
           W_e1, b_e1, W_e2, b_e2, b_conv, W_ih, b_ih, W_hh, b_hh,
           W_d1, b_d1, W_d2, b_d2):
    nf = jnp.pad(node_feats, ((0, NPAD - N), (0, 0)))
    ef = jnp.pad(edge_feats, ((0, EPAD - E), (0, 0)))
    src = jnp.pad(edge_index[0].astype(jnp.int32), (0, EPAD - E))
    dst = jnp.pad(edge_index[1].astype(jnp.int32), (0, EPAD - E),
                  constant_values=N)
    src2d = src.reshape(EPAD // CH, CH)
    dst2d = dst.reshape(EPAD // CH, CH)
    zeros_acc = jnp.zeros((NPAD, W), jnp.float32)
    lane = jnp.arange(H * H, dtype=jnp.int32) // H
    sel = (jnp.arange(H, dtype=jnp.int32)[:, None] == lane[None, :]
           ).astype(jnp.float32)

    x0 = _proj(nf, W_p1, b_p1.reshape(1, H), W_p2, b_p2.reshape(1, H))
    ew = _edgenet(ef, W_e1, b_e1.reshape(1, EH), W_e2.astype(jnp.bfloat16),
                  b_e2.reshape(1, H * H))

    wih = W_ih.T
    whh = W_hh.T
    bih = b_ih.reshape(1, 3 * H)
    bhh = b_hh.reshape(1, 3 * H)
    bconv = b_conv.reshape(1, H)

    nrow_h = EPADH // CH
    srcA = src2d[:nrow_h].reshape(NW, NCHH, CH)
    srcB = src2d[nrow_h:].reshape(NW, NCHH, CH)
    dstA = dst2d[:nrow_h].reshape(NW, NCHH, CH)
    dstB = dst2d[nrow_h:].reshape(NW, NCHH, CH)

    sc_gather, sc_scatter = _sc_kernels()

    def step(hidden, _):
        x = hidden
        zA = sc_gather(x, srcA)
        msgA = _mul(ew, zA, sel, 0)
        zB = sc_gather(x, srcB)
        msgB = _mul(ew, zB, sel, 1)
        aggA = sc_scatter(msgA, dstA, zeros_acc)
        aggB = sc_scatter(msgB, dstB, zeros_acc)
        return _gru(aggA, aggB, hidden, wih, bih, whh, bhh, bconv), None

    x, _ = lax.scan(step, x0, None, length=STEPS)

    out = _dec(x, W_d1, b_d1.reshape(1, H), W_d2, b_d2.reshape(1, D_OUT))
    return out[:N]


# speedup vs baseline: 1.2168x; 1.1748x over previous
import functools

import jax
import jax.numpy as jnp
from jax import lax
from jax.experimental import pallas as pl
from jax.experimental.pallas import tpu as pltpu
from jax.experimental.pallas import tpu_sc as plsc

N = 10000
E = 160000
D_IN = 128
D_E = 16
H = 32
EH = 128
D_OUT = 64
STEPS = 6

W = 128
NPAD = 10240
EPAD = 163840
NC = 2
NS = 16
NW = NC * NS
EPW = EPAD // NW
CH = 128
NCH = EPW // CH
G = 2
GE = G * CH
NG = NCH // G
NSCG = NCH
EPADH = EPAD // 2
EPWH = EPADH // NW
NCHH = EPWH // CH
NGH = NCHH // G
ROWS_PER_SUB = NPAD // NS
ROWS4_PER_SUB = NPAD // 4 // NS


def _proj_body(nf_ref, w1_ref, b1_ref, w2_ref, b2_ref, out_ref):
    h = jnp.maximum(jnp.dot(nf_ref[...], w1_ref[...],
                            preferred_element_type=jnp.float32) + b1_ref[...], 0.0)
    x = jnp.dot(h, w2_ref[...], preferred_element_type=jnp.float32) + b2_ref[...]
    out_ref[...] = jnp.concatenate(
        [x, jnp.zeros((x.shape[0], W - H), jnp.float32)], axis=1)


def _edgenet_body(ef_ref, w1_ref, b1_ref, w2_ref, b2_ref, out_ref):
    h = jnp.maximum(jnp.dot(ef_ref[...], w1_ref[...],
                            preferred_element_type=jnp.float32) + b1_ref[...], 0.0)
    ew = jax.lax.dot_general(h.astype(jnp.bfloat16), w2_ref[...],
                             (((1,), (0,)), ((), ())),
                             preferred_element_type=jnp.float32) + b2_ref[...]
    out_ref[...] = ew.astype(jnp.bfloat16)


def _mul_body(ew_ref, z_ref, sel_ref, out_ref):
    ew = ew_ref[...].astype(jnp.float32)
    zr = jnp.dot(z_ref[:, 0:H], sel_ref[...], preferred_element_type=jnp.float32)
    m = ew * zr
    w = H * H
    while w > H:
        w //= 2
        m = m[:, :w] + m[:, w:2 * w]
    out_ref[...] = jnp.concatenate(
        [m, jnp.zeros((m.shape[0], W - H), jnp.float32)], axis=1)


def _gru_body(agga_ref, aggb_ref, hid_ref, wih_ref, bih_ref, whh_ref,
              bhh_ref, bconv_ref, out_ref):
    agg = (agga_ref[0, :, 0:H] + agga_ref[1, :, 0:H]
           + aggb_ref[0, :, 0:H] + aggb_ref[1, :, 0:H] + bconv_ref[...])
    x = jnp.maximum(agg, 0.0)
    hid = hid_ref[:, 0:H]
    gi = jnp.dot(x, wih_ref[...], preferred_element_type=jnp.float32) + bih_ref[...]
    gh = jnp.dot(hid, whh_ref[...], preferred_element_type=jnp.float32) + bhh_ref[...]
    r = jax.nn.sigmoid(gi[:, 0:H] + gh[:, 0:H])
    zg = jax.nn.sigmoid(gi[:, H:2 * H] + gh[:, H:2 * H])
    n = jnp.tanh(gi[:, 2 * H:3 * H] + r * gh[:, 2 * H:3 * H])
    h_new = (1.0 - zg) * n + zg * hid
    out_ref[...] = jnp.concatenate(
        [h_new, jnp.zeros((h_new.shape[0], W - H), jnp.float32)], axis=1)


def _dec_body(x_ref, w1_ref, b1_ref, w2_ref, b2_ref, out_ref):
    h = jnp.maximum(jnp.dot(x_ref[:, 0:H], w1_ref[...],
                            preferred_element_type=jnp.float32) + b1_ref[...], 0.0)
    out_ref[...] = jnp.dot(h, w2_ref[...],
                           preferred_element_type=jnp.float32) + b2_ref[...]


def _full(shape):
    return pl.BlockSpec(shape, lambda *_: tuple(0 for _ in shape))


def _proj(nf, w1, b1, w2, b2):
    return pl.pallas_call(
        _proj_body,
        grid=(1,),
        in_specs=[_full((NPAD, D_IN)), _full((D_IN, H)), _full((1, H)),
                  _full((H, H)), _full((1, H))],
        out_specs=_full((NPAD, W)),
        out_shape=jax.ShapeDtypeStruct((NPAD, W), jnp.float32),
    )(nf, w1, b1, w2, b2)


_ET = 1024


def _edgenet(ef, w1, b1, w2, b2):
    return pl.pallas_call(
        _edgenet_body,
        grid=(EPAD // _ET,),
        in_specs=[pl.BlockSpec((_ET, D_E), lambda i: (i, 0)),
                  _full((D_E, EH)), _full((1, EH)),
                  _full((EH, H * H)), _full((1, H * H))],
        out_specs=pl.BlockSpec((_ET, H * H), lambda i: (i, 0)),
        out_shape=jax.ShapeDtypeStruct((EPAD, H * H), jnp.bfloat16),
    )(ef, w1, b1, w2, b2)


def _mul(ew, z, sel, half):
    toff = half * (EPADH // _ET)
    return pl.pallas_call(
        _mul_body,
        grid=(EPADH // _ET,),
        in_specs=[pl.BlockSpec((_ET, H * H), lambda i, t=toff: (i + t, 0)),
                  pl.BlockSpec((_ET, W), lambda i: (i, 0)),
                  _full((H, H * H))],
        out_specs=pl.BlockSpec((_ET, W), lambda i: (i, 0)),
        out_shape=jax.ShapeDtypeStruct((EPADH, W), jnp.float32),
    )(ew, z, sel)


def _gru(agga, aggb, hid, wih, bih, whh, bhh, bconv):
    return pl.pallas_call(
        _gru_body,
        grid=(1,),
        in_specs=[_full((2, NPAD, W)), _full((2, NPAD, W)), _full((NPAD, W)),
                  _full((H, 3 * H)), _full((1, 3 * H)),
                  _full((H, 3 * H)), _full((1, 3 * H)), _full((1, H))],
        out_specs=_full((NPAD, W)),
        out_shape=jax.ShapeDtypeStruct((NPAD, W), jnp.float32),
    )(agga, aggb, hid, wih, bih, whh, bhh, bconv)


def _dec(x, w1, b1, w2, b2):
    return pl.pallas_call(
        _dec_body,
        grid=(1,),
        in_specs=[_full((NPAD, W)), _full((H, H)), _full((1, H)),
                  _full((H, D_OUT)), _full((1, D_OUT))],
        out_specs=_full((NPAD, D_OUT)),
        out_shape=jax.ShapeDtypeStruct((NPAD, D_OUT), jnp.float32),
    )(x, w1, b1, w2, b2)


def _sc_gather_body(x_hbm, src_hbm, out_hbm, idx_v, buf, sem_g, sem_w):
    c = lax.axis_index("c")
    s = lax.axis_index("s")
    w = c * NS + s
    base = w * EPWH
    pltpu.sync_copy(src_hbm.at[w], idx_v)

    def fire_gather(g, b):
        for q in range(G):
            pltpu.async_copy(x_hbm.at[idx_v.at[g * G + q]],
                             buf.at[b, pl.ds(q * CH, CH)], sem_g)

    def wait_gather(b):
        pltpu.make_async_copy(x_hbm.at[pl.ds(0, GE)], buf.at[b], sem_g).wait()

    def fire_write(g, b):
        pltpu.async_copy(buf.at[b], out_hbm.at[pl.ds(base + g * GE, GE)], sem_w)

    def wait_one_write(b):
        pltpu.make_async_copy(buf.at[b],
                              out_hbm.at[pl.ds(base, GE)], sem_w).wait()

    fire_gather(0, 0)
    fire_gather(1, 1)

    def pair(p, carry):
        g0 = p * 2
        wait_gather(0)
        fire_write(g0, 0)
        wait_one_write(0)
        fire_gather(g0 + 2, 0)
        wait_gather(1)
        fire_write(g0 + 1, 1)
        wait_one_write(1)
        fire_gather(g0 + 3, 1)
        return carry

    lax.fori_loop(0, NGH // 2 - 1, pair, 0)
    wait_gather(0)
    fire_write(NGH - 2, 0)
    wait_gather(1)
    fire_write(NGH - 1, 1)
    wait_one_write(0)
    wait_one_write(1)


def _sc_scatter_body(msg_hbm, dst_hbm, zeros_hbm, out_hbm, idx_v, buf, acc_sh,
                     sem_l, sem_a):
    c = lax.axis_index("c")
    s = lax.axis_index("s")
    w = c * NS + s
    base = w * EPWH

    def fire_load(g, b):
        pltpu.async_copy(msg_hbm.at[pl.ds(base + g * CH, CH)], buf.at[b], sem_l)

    def wait_load(b):
        pltpu.make_async_copy(msg_hbm.at[pl.ds(base, CH)], buf.at[b],
                              sem_l).wait()

    def fire_add(g, b):
        pltpu.async_copy(buf.at[b], acc_sh.at[idx_v.at[g]], sem_a, add=True)

    def wait_one_add(b):
        pltpu.make_async_copy(buf.at[b], acc_sh.at[pl.ds(0, CH)], sem_a).wait()

    fire_load(0, 0)
    fire_load(1, 1)
    pltpu.sync_copy(zeros_hbm.at[pl.ds(s * ROWS_PER_SUB, ROWS_PER_SUB)],
                    acc_sh.at[pl.ds(s * ROWS_PER_SUB, ROWS_PER_SUB)])
    pltpu.sync_copy(dst_hbm.at[w], idx_v)
    plsc.subcore_barrier()

    def pair(p, carry):
        g0 = p * 2
        wait_load(0)
        fire_add(g0, 0)
        wait_load(1)
        fire_add(g0 + 1, 1)
        wait_one_add(0)
        fire_load(g0 + 2, 0)
        wait_one_add(1)
        fire_load(g0 + 3, 1)
        return carry

    lax.fori_loop(0, NCHH // 2 - 1, pair, 0)
    wait_load(0)
    fire_add(NCHH - 2, 0)
    wait_load(1)
    fire_add(NCHH - 1, 1)
    wait_one_add(0)
    wait_one_add(1)
    plsc.subcore_barrier()
    pltpu.sync_copy(acc_sh.at[pl.ds(s * ROWS_PER_SUB, ROWS_PER_SUB)],
                    out_hbm.at[c].at[pl.ds(s * ROWS_PER_SUB, ROWS_PER_SUB)])


@functools.cache
def _sc_kernels():
    mesh = plsc.VectorSubcoreMesh(core_axis_name="c", subcore_axis_name="s",
                                  num_cores=NC, num_subcores=NS)
    gather = pl.kernel(
        _sc_gather_body,
        out_type=jax.ShapeDtypeStruct((EPADH, W), jnp.float32),
        mesh=mesh,
        scratch_types=[pltpu.VMEM((NCHH, CH), jnp.int32),
                       pltpu.VMEM((2, GE, W), jnp.float32),
                       pltpu.SemaphoreType.DMA,
                       pltpu.SemaphoreType.DMA],
    )
    scatter = pl.kernel(
        _sc_scatter_body,
        out_type=jax.ShapeDtypeStruct((NC, NPAD, W), jnp.float32),
        mesh=mesh,
        scratch_types=[pltpu.VMEM((NCHH, CH), jnp.int32),
                       pltpu.VMEM((2, CH, W), jnp.float32),
                       pltpu.VMEM_SHARED((NPAD, W), jnp.float32),
                       pltpu.SemaphoreType.DMA,
                       pltpu.SemaphoreType.DMA],
    )
    return gather, scatter


def kernel(node_feats, edge_feats, edge_index, W_p1, b_p1, W_p2, b_p2,
           W_e1, b_e1, W_e2, b_e2, b_conv, W_ih, b_ih, W_hh, b_hh,
           W_d1, b_d1, W_d2, b_d2):
    nf = jnp.pad(node_feats, ((0, NPAD - N), (0, 0)))
    ef = jnp.pad(edge_feats, ((0, EPAD - E), (0, 0)))
    src = jnp.pad(edge_index[0].astype(jnp.int32), (0, EPAD - E))
    dst = jnp.pad(edge_index[1].astype(jnp.int32), (0, EPAD - E),
                  constant_values=N)
    src2d = src.reshape(EPAD // CH, CH)
    dst2d = dst.reshape(EPAD // CH, CH)
    zeros_acc = jnp.zeros((NPAD, W), jnp.float32)
    lane = jnp.arange(H * H, dtype=jnp.int32) // H
    sel = (jnp.arange(H, dtype=jnp.int32)[:, None] == lane[None, :]
           ).astype(jnp.float32)

    x0 = _proj(nf, W_p1, b_p1.reshape(1, H), W_p2, b_p2.reshape(1, H))
    ew = _edgenet(ef, W_e1, b_e1.reshape(1, EH), W_e2.astype(jnp.bfloat16),
                  b_e2.reshape(1, H * H))

    wih = W_ih.T
    whh = W_hh.T
    bih = b_ih.reshape(1, 3 * H)
    bhh = b_hh.reshape(1, 3 * H)
    bconv = b_conv.reshape(1, H)

    nrow_h = EPADH // CH
    srcA = src2d[:nrow_h].reshape(NW, NCHH, CH)
    srcB = src2d[nrow_h:].reshape(NW, NCHH, CH)
    dstA = dst2d[:nrow_h].reshape(NW, NCHH, CH)
    dstB = dst2d[nrow_h:].reshape(NW, NCHH, CH)

    sc_gather, sc_scatter = _sc_kernels()

    def step(hidden, _):
        x = hidden
        zA = sc_gather(x, srcA)
        msgA = _mul(ew, zA, sel, 0)
        zB = sc_gather(x, srcB)
        msgB = _mul(ew, zB, sel, 1)
        aggA = sc_scatter(msgA, dstA, zeros_acc)
        aggB = sc_scatter(msgB, dstB, zeros_acc)
        return _gru(aggA, aggB, hidden, wih, bih, whh, bhh, bconv), None

    x, _ = lax.scan(step, x0, None, length=STEPS)

    out = _dec(x, W_d1, b_d1.reshape(1, H), W_d2, b_d2.reshape(1, D_OUT))
    return out[:N]


# speedup vs baseline: 1.3321x; 1.0948x over previous
import functools

import jax
import jax.numpy as jnp
from jax import lax
from jax.experimental import pallas as pl
from jax.experimental.pallas import tpu as pltpu
from jax.experimental.pallas import tpu_sc as plsc

N = 10000
E = 160000
D_IN = 128
D_E = 16
H = 32
EH = 128
D_OUT = 64
STEPS = 6

W = 128
NPAD = 10240
EPAD = 163840
NC = 2
NS = 16
NW = NC * NS
EPW = EPAD // NW
CH = 128
NCH = EPW // CH
G = 2
GE = G * CH
NG = NCH // G
NSCG = NCH
EPADH = EPAD // 2
EPWH = EPADH // NW
NCHH = EPWH // CH
NGH = NCHH // G
ROWS_PER_SUB = NPAD // NS
ROWS4_PER_SUB = NPAD // 4 // NS


def _proj_body(nf_ref, w1_ref, b1_ref, w2_ref, b2_ref, out_ref):
    h = jnp.maximum(jnp.dot(nf_ref[...], w1_ref[...],
                            preferred_element_type=jnp.float32) + b1_ref[...], 0.0)
    x = jnp.dot(h, w2_ref[...], preferred_element_type=jnp.float32) + b2_ref[...]
    out_ref[...] = jnp.concatenate(
        [x, jnp.zeros((x.shape[0], W - H), jnp.float32)], axis=1)


def _edgenet_body(ef_ref, w1_ref, b1_ref, w2_ref, b2_ref, out_ref):
    h = jnp.maximum(jnp.dot(ef_ref[...], w1_ref[...],
                            preferred_element_type=jnp.float32) + b1_ref[...], 0.0)
    ew = jax.lax.dot_general(h.astype(jnp.bfloat16), w2_ref[...],
                             (((1,), (0,)), ((), ())),
                             preferred_element_type=jnp.float32) + b2_ref[...]
    out_ref[...] = ew.astype(jnp.bfloat16)


def _mul_body(ew_ref, z_ref, sel_ref, out_ref):
    ew = ew_ref[...].astype(jnp.float32)
    zr = jnp.dot(z_ref[:, 0:H], sel_ref[...], preferred_element_type=jnp.float32)
    m = ew * zr
    w = H * H
    while w > H:
        w //= 2
        m = m[:, :w] + m[:, w:2 * w]
    out_ref[...] = jnp.concatenate(
        [m, jnp.zeros((m.shape[0], W - H), jnp.float32)], axis=1)


def _gru_body(agga_ref, aggb_ref, hid_ref, wih_ref, bih_ref, whh_ref,
              bhh_ref, bconv_ref, out_ref):
    agg = (agga_ref[0, :, 0:H] + agga_ref[1, :, 0:H]
           + aggb_ref[0, :, 0:H] + aggb_ref[1, :, 0:H] + bconv_ref[...])
    x = jnp.maximum(agg, 0.0)
    hid = hid_ref[:, 0:H]
    gi = jnp.dot(x, wih_ref[...], preferred_element_type=jnp.float32) + bih_ref[...]
    gh = jnp.dot(hid, whh_ref[...], preferred_element_type=jnp.float32) + bhh_ref[...]
    r = jax.nn.sigmoid(gi[:, 0:H] + gh[:, 0:H])
    zg = jax.nn.sigmoid(gi[:, H:2 * H] + gh[:, H:2 * H])
    n = jnp.tanh(gi[:, 2 * H:3 * H] + r * gh[:, 2 * H:3 * H])
    h_new = (1.0 - zg) * n + zg * hid
    out_ref[...] = jnp.concatenate(
        [h_new, jnp.zeros((h_new.shape[0], W - H), jnp.float32)], axis=1)


def _dec_body(x_ref, w1_ref, b1_ref, w2_ref, b2_ref, out_ref):
    h = jnp.maximum(jnp.dot(x_ref[:, 0:H], w1_ref[...],
                            preferred_element_type=jnp.float32) + b1_ref[...], 0.0)
    out_ref[...] = jnp.dot(h, w2_ref[...],
                           preferred_element_type=jnp.float32) + b2_ref[...]


def _full(shape):
    return pl.BlockSpec(shape, lambda *_: tuple(0 for _ in shape))


def _proj(nf, w1, b1, w2, b2):
    return pl.pallas_call(
        _proj_body,
        grid=(1,),
        in_specs=[_full((NPAD, D_IN)), _full((D_IN, H)), _full((1, H)),
                  _full((H, H)), _full((1, H))],
        out_specs=_full((NPAD, W)),
        out_shape=jax.ShapeDtypeStruct((NPAD, W), jnp.float32),
    )(nf, w1, b1, w2, b2)


_ET = 2048


def _edgenet(ef, w1, b1, w2, b2):
    return pl.pallas_call(
        _edgenet_body,
        grid=(EPAD // _ET,),
        in_specs=[pl.BlockSpec((_ET, D_E), lambda i: (i, 0)),
                  _full((D_E, EH)), _full((1, EH)),
                  _full((EH, H * H)), _full((1, H * H))],
        out_specs=pl.BlockSpec((_ET, H * H), lambda i: (i, 0)),
        out_shape=jax.ShapeDtypeStruct((EPAD, H * H), jnp.bfloat16),
    )(ef, w1, b1, w2, b2)


def _mul(ew, z, sel, half):
    toff = half * (EPADH // _ET)
    return pl.pallas_call(
        _mul_body,
        grid=(EPADH // _ET,),
        in_specs=[pl.BlockSpec((_ET, H * H), lambda i, t=toff: (i + t, 0)),
                  pl.BlockSpec((_ET, W), lambda i: (i, 0)),
                  _full((H, H * H))],
        out_specs=pl.BlockSpec((_ET, W), lambda i: (i, 0)),
        out_shape=jax.ShapeDtypeStruct((EPADH, W), jnp.float32),
    )(ew, z, sel)


def _gru(agga, aggb, hid, wih, bih, whh, bhh, bconv):
    return pl.pallas_call(
        _gru_body,
        grid=(1,),
        in_specs=[_full((2, NPAD, W)), _full((2, NPAD, W)), _full((NPAD, W)),
                  _full((H, 3 * H)), _full((1, 3 * H)),
                  _full((H, 3 * H)), _full((1, 3 * H)), _full((1, H))],
        out_specs=_full((NPAD, W)),
        out_shape=jax.ShapeDtypeStruct((NPAD, W), jnp.float32),
    )(agga, aggb, hid, wih, bih, whh, bhh, bconv)


def _dec(x, w1, b1, w2, b2):
    return pl.pallas_call(
        _dec_body,
        grid=(1,),
        in_specs=[_full((NPAD, W)), _full((H, H)), _full((1, H)),
                  _full((H, D_OUT)), _full((1, D_OUT))],
        out_specs=_full((NPAD, D_OUT)),
        out_shape=jax.ShapeDtypeStruct((NPAD, D_OUT), jnp.float32),
    )(x, w1, b1, w2, b2)


def _sc_gather_body(x_hbm, src_hbm, out_hbm, idx_v, buf, sem_g, sem_w):
    c = lax.axis_index("c")
    s = lax.axis_index("s")
    w = c * NS + s
    base = w * EPWH
    pltpu.sync_copy(src_hbm.at[w], idx_v)

    def fire_gather(g, b):
        for q in range(G):
            pltpu.async_copy(x_hbm.at[idx_v.at[g * G + q]],
                             buf.at[b, pl.ds(q * CH, CH)], sem_g)

    def wait_gather(b):
        pltpu.make_async_copy(x_hbm.at[pl.ds(0, GE)], buf.at[b], sem_g).wait()

    def fire_write(g, b):
        pltpu.async_copy(buf.at[b], out_hbm.at[pl.ds(base + g * GE, GE)], sem_w)

    def wait_one_write(b):
        pltpu.make_async_copy(buf.at[b],
                              out_hbm.at[pl.ds(base, GE)], sem_w).wait()

    fire_gather(0, 0)
    fire_gather(1, 1)

    def pair(p, carry):
        g0 = p * 2
        wait_gather(0)
        fire_write(g0, 0)
        wait_one_write(0)
        fire_gather(g0 + 2, 0)
        wait_gather(1)
        fire_write(g0 + 1, 1)
        wait_one_write(1)
        fire_gather(g0 + 3, 1)
        return carry

    lax.fori_loop(0, NGH // 2 - 1, pair, 0)
    wait_gather(0)
    fire_write(NGH - 2, 0)
    wait_gather(1)
    fire_write(NGH - 1, 1)
    wait_one_write(0)
    wait_one_write(1)


def _sc_scatter_body(msg_hbm, dst_hbm, zeros_hbm, out_hbm, idx_v, buf, acc_sh,
                     sem_l, sem_a):
    c = lax.axis_index("c")
    s = lax.axis_index("s")
    w = c * NS + s
    base = w * EPWH

    def fire_load(g, b):
        pltpu.async_copy(msg_hbm.at[pl.ds(base + g * CH, CH)], buf.at[b], sem_l)

    def wait_load(b):
        pltpu.make_async_copy(msg_hbm.at[pl.ds(base, CH)], buf.at[b],
                              sem_l).wait()

    def fire_add(g, b):
        pltpu.async_copy(buf.at[b], acc_sh.at[idx_v.at[g]], sem_a, add=True)

    def wait_one_add(b):
        pltpu.make_async_copy(buf.at[b], acc_sh.at[pl.ds(0, CH)], sem_a).wait()

    fire_load(0, 0)
    fire_load(1, 1)
    pltpu.sync_copy(zeros_hbm.at[pl.ds(s * ROWS_PER_SUB, ROWS_PER_SUB)],
                    acc_sh.at[pl.ds(s * ROWS_PER_SUB, ROWS_PER_SUB)])
    pltpu.sync_copy(dst_hbm.at[w], idx_v)
    plsc.subcore_barrier()

    def pair(p, carry):
        g0 = p * 2
        wait_load(0)
        fire_add(g0, 0)
        wait_load(1)
        fire_add(g0 + 1, 1)
        wait_one_add(0)
        fire_load(g0 + 2, 0)
        wait_one_add(1)
        fire_load(g0 + 3, 1)
        return carry

    lax.fori_loop(0, NCHH // 2 - 1, pair, 0)
    wait_load(0)
    fire_add(NCHH - 2, 0)
    wait_load(1)
    fire_add(NCHH - 1, 1)
    wait_one_add(0)
    wait_one_add(1)
    plsc.subcore_barrier()
    pltpu.sync_copy(acc_sh.at[pl.ds(s * ROWS_PER_SUB, ROWS_PER_SUB)],
                    out_hbm.at[c].at[pl.ds(s * ROWS_PER_SUB, ROWS_PER_SUB)])


@functools.cache
def _sc_kernels():
    mesh = plsc.VectorSubcoreMesh(core_axis_name="c", subcore_axis_name="s",
                                  num_cores=NC, num_subcores=NS)
    gather = pl.kernel(
        _sc_gather_body,
        out_type=jax.ShapeDtypeStruct((EPADH, W), jnp.float32),
        mesh=mesh,
        scratch_types=[pltpu.VMEM((NCHH, CH), jnp.int32),
                       pltpu.VMEM((2, GE, W), jnp.float32),
                       pltpu.SemaphoreType.DMA,
                       pltpu.SemaphoreType.DMA],
    )
    scatter = pl.kernel(
        _sc_scatter_body,
        out_type=jax.ShapeDtypeStruct((NC, NPAD, W), jnp.float32),
        mesh=mesh,
        scratch_types=[pltpu.VMEM((NCHH, CH), jnp.int32),
                       pltpu.VMEM((2, CH, W), jnp.float32),
                       pltpu.VMEM_SHARED((NPAD, W), jnp.float32),
                       pltpu.SemaphoreType.DMA,
                       pltpu.SemaphoreType.DMA],
    )
    return gather, scatter


def kernel(node_feats, edge_feats, edge_index, W_p1, b_p1, W_p2, b_p2,
           W_e1, b_e1, W_e2, b_e2, b_conv, W_ih, b_ih, W_hh, b_hh,
           W_d1, b_d1, W_d2, b_d2):
    nf = jnp.pad(node_feats, ((0, NPAD - N), (0, 0)))
    ef = jnp.pad(edge_feats, ((0, EPAD - E), (0, 0)))
    src = jnp.pad(edge_index[0].astype(jnp.int32), (0, EPAD - E))
    dst = jnp.pad(edge_index[1].astype(jnp.int32), (0, EPAD - E),
                  constant_values=N)
    src2d = src.reshape(EPAD // CH, CH)
    dst2d = dst.reshape(EPAD // CH, CH)
    zeros_acc = jnp.zeros((NPAD, W), jnp.float32)
    lane = jnp.arange(H * H, dtype=jnp.int32) // H
    sel = (jnp.arange(H, dtype=jnp.int32)[:, None] == lane[None, :]
           ).astype(jnp.float32)

    x0 = _proj(nf, W_p1, b_p1.reshape(1, H), W_p2, b_p2.reshape(1, H))
    ew = _edgenet(ef, W_e1, b_e1.reshape(1, EH), W_e2.astype(jnp.bfloat16),
                  b_e2.reshape(1, H * H))

    wih = W_ih.T
    whh = W_hh.T
    bih = b_ih.reshape(1, 3 * H)
    bhh = b_hh.reshape(1, 3 * H)
    bconv = b_conv.reshape(1, H)

    nrow_h = EPADH // CH
    srcA = src2d[:nrow_h].reshape(NW, NCHH, CH)
    srcB = src2d[nrow_h:].reshape(NW, NCHH, CH)
    dstA = dst2d[:nrow_h].reshape(NW, NCHH, CH)
    dstB = dst2d[nrow_h:].reshape(NW, NCHH, CH)

    sc_gather, sc_scatter = _sc_kernels()

    def step(hidden, _):
        x = hidden
        zA = sc_gather(x, srcA)
        msgA = _mul(ew, zA, sel, 0)
        zB = sc_gather(x, srcB)
        msgB = _mul(ew, zB, sel, 1)
        aggA = sc_scatter(msgA, dstA, zeros_acc)
        aggB = sc_scatter(msgB, dstB, zeros_acc)
        return _gru(aggA, aggB, hidden, wih, bih, whh, bhh, bconv), None

    x, _ = lax.scan(step, x0, None, length=STEPS)

    out = _dec(x, W_d1, b_d1.reshape(1, H), W_d2, b_d2.reshape(1, D_OUT))
    return out[:N]


# speedup vs baseline: 1.3863x; 1.0407x over previous
import functools

import jax
import jax.numpy as jnp
from jax import lax
from jax.experimental import pallas as pl
from jax.experimental.pallas import tpu as pltpu
from jax.experimental.pallas import tpu_sc as plsc

N = 10000
E = 160000
D_IN = 128
D_E = 16
H = 32
EH = 128
D_OUT = 64
STEPS = 6

W = 128
NPAD = 10240
EPAD = 163840
NC = 2
NS = 16
NW = NC * NS
EPW = EPAD // NW
CH = 128
NCH = EPW // CH
G = 2
GE = G * CH
NG = NCH // G
NSCG = NCH
EPADH = EPAD // 2
EPWH = EPADH // NW
NCHH = EPWH // CH
NGH = NCHH // G
ROWS_PER_SUB = NPAD // NS
ROWS4_PER_SUB = NPAD // 4 // NS


def _proj_body(nf_ref, w1_ref, b1_ref, w2_ref, b2_ref, out_ref):
    h = jnp.maximum(jnp.dot(nf_ref[...], w1_ref[...],
                            preferred_element_type=jnp.float32) + b1_ref[...], 0.0)
    x = jnp.dot(h, w2_ref[...], preferred_element_type=jnp.float32) + b2_ref[...]
    out_ref[...] = jnp.concatenate(
        [x, jnp.zeros((x.shape[0], W - H), jnp.float32)], axis=1)


def _edgenet_body(ef_ref, w1_ref, b1_ref, w2_ref, b2_ref, out_ref):
    h = jnp.maximum(jnp.dot(ef_ref[...], w1_ref[...],
                            preferred_element_type=jnp.float32) + b1_ref[...], 0.0)
    ew = jax.lax.dot_general(h.astype(jnp.bfloat16), w2_ref[...],
                             (((1,), (0,)), ((), ())),
                             preferred_element_type=jnp.float32) + b2_ref[...]
    out_ref[...] = ew.astype(jnp.bfloat16)


def _mul_body(ew_ref, z_ref, sel_ref, out_ref):
    ew = ew_ref[...].astype(jnp.float32)
    zr = jnp.dot(z_ref[:, 0:H], sel_ref[...], preferred_element_type=jnp.float32)
    m = ew * zr
    w = H * H
    while w > H:
        w //= 2
        m = m[:, :w] + m[:, w:2 * w]
    out_ref[...] = jnp.concatenate(
        [m, jnp.zeros((m.shape[0], W - H), jnp.float32)], axis=1)


def _gru_body(agga_ref, aggb_ref, hid_ref, wih_ref, bih_ref, whh_ref,
              bhh_ref, bconv_ref, out_ref):
    agg = (agga_ref[0, :, 0:H] + agga_ref[1, :, 0:H]
           + aggb_ref[0, :, 0:H] + aggb_ref[1, :, 0:H] + bconv_ref[...])
    x = jnp.maximum(agg, 0.0)
    hid = hid_ref[:, 0:H]
    gi = jnp.dot(x, wih_ref[...], preferred_element_type=jnp.float32) + bih_ref[...]
    gh = jnp.dot(hid, whh_ref[...], preferred_element_type=jnp.float32) + bhh_ref[...]
    r = jax.nn.sigmoid(gi[:, 0:H] + gh[:, 0:H])
    zg = jax.nn.sigmoid(gi[:, H:2 * H] + gh[:, H:2 * H])
    n = jnp.tanh(gi[:, 2 * H:3 * H] + r * gh[:, 2 * H:3 * H])
    h_new = (1.0 - zg) * n + zg * hid
    out_ref[...] = jnp.concatenate(
        [h_new, jnp.zeros((h_new.shape[0], W - H), jnp.float32)], axis=1)


def _dec_body(x_ref, w1_ref, b1_ref, w2_ref, b2_ref, out_ref):
    h = jnp.maximum(jnp.dot(x_ref[:, 0:H], w1_ref[...],
                            preferred_element_type=jnp.float32) + b1_ref[...], 0.0)
    out_ref[...] = jnp.dot(h, w2_ref[...],
                           preferred_element_type=jnp.float32) + b2_ref[...]


def _full(shape):
    return pl.BlockSpec(shape, lambda *_: tuple(0 for _ in shape))


def _proj(nf, w1, b1, w2, b2):
    return pl.pallas_call(
        _proj_body,
        grid=(1,),
        in_specs=[_full((NPAD, D_IN)), _full((D_IN, H)), _full((1, H)),
                  _full((H, H)), _full((1, H))],
        out_specs=_full((NPAD, W)),
        out_shape=jax.ShapeDtypeStruct((NPAD, W), jnp.float32),
    )(nf, w1, b1, w2, b2)


_ET = 4096


def _edgenet(ef, w1, b1, w2, b2):
    return pl.pallas_call(
        _edgenet_body,
        grid=(EPAD // _ET,),
        in_specs=[pl.BlockSpec((_ET, D_E), lambda i: (i, 0)),
                  _full((D_E, EH)), _full((1, EH)),
                  _full((EH, H * H)), _full((1, H * H))],
        out_specs=pl.BlockSpec((_ET, H * H), lambda i: (i, 0)),
        out_shape=jax.ShapeDtypeStruct((EPAD, H * H), jnp.bfloat16),
    )(ef, w1, b1, w2, b2)


def _mul(ew, z, sel, half):
    toff = half * (EPADH // _ET)
    return pl.pallas_call(
        _mul_body,
        grid=(EPADH // _ET,),
        in_specs=[pl.BlockSpec((_ET, H * H), lambda i, t=toff: (i + t, 0)),
                  pl.BlockSpec((_ET, W), lambda i: (i, 0)),
                  _full((H, H * H))],
        out_specs=pl.BlockSpec((_ET, W), lambda i: (i, 0)),
        out_shape=jax.ShapeDtypeStruct((EPADH, W), jnp.float32),
    )(ew, z, sel)


def _gru(agga, aggb, hid, wih, bih, whh, bhh, bconv):
    return pl.pallas_call(
        _gru_body,
        grid=(1,),
        in_specs=[_full((2, NPAD, W)), _full((2, NPAD, W)), _full((NPAD, W)),
                  _full((H, 3 * H)), _full((1, 3 * H)),
                  _full((H, 3 * H)), _full((1, 3 * H)), _full((1, H))],
        out_specs=_full((NPAD, W)),
        out_shape=jax.ShapeDtypeStruct((NPAD, W), jnp.float32),
    )(agga, aggb, hid, wih, bih, whh, bhh, bconv)


def _dec(x, w1, b1, w2, b2):
    return pl.pallas_call(
        _dec_body,
        grid=(1,),
        in_specs=[_full((NPAD, W)), _full((H, H)), _full((1, H)),
                  _full((H, D_OUT)), _full((1, D_OUT))],
        out_specs=_full((NPAD, D_OUT)),
        out_shape=jax.ShapeDtypeStruct((NPAD, D_OUT), jnp.float32),
    )(x, w1, b1, w2, b2)


def _sc_gather_body(x_hbm, src_hbm, out_hbm, idx_v, buf, sem_g, sem_w):
    c = lax.axis_index("c")
    s = lax.axis_index("s")
    w = c * NS + s
    base = w * EPWH
    pltpu.sync_copy(src_hbm.at[w], idx_v)

    def fire_gather(g, b):
        for q in range(G):
            pltpu.async_copy(x_hbm.at[idx_v.at[g * G + q]],
                             buf.at[b, pl.ds(q * CH, CH)], sem_g)

    def wait_gather(b):
        pltpu.make_async_copy(x_hbm.at[pl.ds(0, GE)], buf.at[b], sem_g).wait()

    def fire_write(g, b):
        pltpu.async_copy(buf.at[b], out_hbm.at[pl.ds(base + g * GE, GE)], sem_w)

    def wait_one_write(b):
        pltpu.make_async_copy(buf.at[b],
                              out_hbm.at[pl.ds(base, GE)], sem_w).wait()

    fire_gather(0, 0)
    fire_gather(1, 1)

    def pair(p, carry):
        g0 = p * 2
        wait_gather(0)
        fire_write(g0, 0)
        wait_one_write(0)
        fire_gather(g0 + 2, 0)
        wait_gather(1)
        fire_write(g0 + 1, 1)
        wait_one_write(1)
        fire_gather(g0 + 3, 1)
        return carry

    lax.fori_loop(0, NGH // 2 - 1, pair, 0)
    wait_gather(0)
    fire_write(NGH - 2, 0)
    wait_gather(1)
    fire_write(NGH - 1, 1)
    wait_one_write(0)
    wait_one_write(1)


def _sc_scatter_body(msg_hbm, dst_hbm, zeros_hbm, out_hbm, idx_v, buf, acc_sh,
                     sem_l, sem_a):
    c = lax.axis_index("c")
    s = lax.axis_index("s")
    w = c * NS + s
    base = w * EPWH

    def fire_load(g, b):
        pltpu.async_copy(msg_hbm.at[pl.ds(base + g * CH, CH)], buf.at[b], sem_l)

    def wait_load(b):
        pltpu.make_async_copy(msg_hbm.at[pl.ds(base, CH)], buf.at[b],
                              sem_l).wait()

    def fire_add(g, b):
        pltpu.async_copy(buf.at[b], acc_sh.at[idx_v.at[g]], sem_a, add=True)

    def wait_one_add(b):
        pltpu.make_async_copy(buf.at[b], acc_sh.at[pl.ds(0, CH)], sem_a).wait()

    fire_load(0, 0)
    fire_load(1, 1)
    pltpu.sync_copy(zeros_hbm.at[pl.ds(s * ROWS_PER_SUB, ROWS_PER_SUB)],
                    acc_sh.at[pl.ds(s * ROWS_PER_SUB, ROWS_PER_SUB)])
    pltpu.sync_copy(dst_hbm.at[w], idx_v)
    plsc.subcore_barrier()

    def pair(p, carry):
        g0 = p * 2
        wait_load(0)
        fire_add(g0, 0)
        wait_load(1)
        fire_add(g0 + 1, 1)
        wait_one_add(0)
        fire_load(g0 + 2, 0)
        wait_one_add(1)
        fire_load(g0 + 3, 1)
        return carry

    lax.fori_loop(0, NCHH // 2 - 1, pair, 0)
    wait_load(0)
    fire_add(NCHH - 2, 0)
    wait_load(1)
    fire_add(NCHH - 1, 1)
    wait_one_add(0)
    wait_one_add(1)
    plsc.subcore_barrier()
    pltpu.sync_copy(acc_sh.at[pl.ds(s * ROWS_PER_SUB, ROWS_PER_SUB)],
                    out_hbm.at[c].at[pl.ds(s * ROWS_PER_SUB, ROWS_PER_SUB)])


@functools.cache
def _sc_kernels():
    mesh = plsc.VectorSubcoreMesh(core_axis_name="c", subcore_axis_name="s",
                                  num_cores=NC, num_subcores=NS)
    gather = pl.kernel(
        _sc_gather_body,
        out_type=jax.ShapeDtypeStruct((EPADH, W), jnp.float32),
        mesh=mesh,
        scratch_types=[pltpu.VMEM((NCHH, CH), jnp.int32),
                       pltpu.VMEM((2, GE, W), jnp.float32),
                       pltpu.SemaphoreType.DMA,
                       pltpu.SemaphoreType.DMA],
    )
    scatter = pl.kernel(
        _sc_scatter_body,
        out_type=jax.ShapeDtypeStruct((NC, NPAD, W), jnp.float32),
        mesh=mesh,
        scratch_types=[pltpu.VMEM((NCHH, CH), jnp.int32),
                       pltpu.VMEM((2, CH, W), jnp.float32),
                       pltpu.VMEM_SHARED((NPAD, W), jnp.float32),
                       pltpu.SemaphoreType.DMA,
                       pltpu.SemaphoreType.DMA],
    )
    return gather, scatter


def kernel(node_feats, edge_feats, edge_index, W_p1, b_p1, W_p2, b_p2,
           W_e1, b_e1, W_e2, b_e2, b_conv, W_ih, b_ih, W_hh, b_hh,
           W_d1, b_d1, W_d2, b_d2):
    nf = jnp.pad(node_feats, ((0, NPAD - N), (0, 0)))
    ef = jnp.pad(edge_feats, ((0, EPAD - E), (0, 0)))
    src = jnp.pad(edge_index[0].astype(jnp.int32), (0, EPAD - E))
    dst = jnp.pad(edge_index[1].astype(jnp.int32), (0, EPAD - E),
                  constant_values=N)
    src2d = src.reshape(EPAD // CH, CH)
    dst2d = dst.reshape(EPAD // CH, CH)
    zeros_acc = jnp.zeros((NPAD, W), jnp.float32)
    lane = jnp.arange(H * H, dtype=jnp.int32) // H
    sel = (jnp.arange(H, dtype=jnp.int32)[:, None] == lane[None, :]
           ).astype(jnp.float32)

    x0 = _proj(nf, W_p1, b_p1.reshape(1, H), W_p2, b_p2.reshape(1, H))
    ew = _edgenet(ef, W_e1, b_e1.reshape(1, EH), W_e2.astype(jnp.bfloat16),
                  b_e2.reshape(1, H * H))

    wih = W_ih.T
    whh = W_hh.T
    bih = b_ih.reshape(1, 3 * H)
    bhh = b_hh.reshape(1, 3 * H)
    bconv = b_conv.reshape(1, H)

    nrow_h = EPADH // CH
    srcA = src2d[:nrow_h].reshape(NW, NCHH, CH)
    srcB = src2d[nrow_h:].reshape(NW, NCHH, CH)
    dstA = dst2d[:nrow_h].reshape(NW, NCHH, CH)
    dstB = dst2d[nrow_h:].reshape(NW, NCHH, CH)

    sc_gather, sc_scatter = _sc_kernels()

    def step(hidden, _):
        x = hidden
        zA = sc_gather(x, srcA)
        msgA = _mul(ew, zA, sel, 0)
        zB = sc_gather(x, srcB)
        msgB = _mul(ew, zB, sel, 1)
        aggA = sc_scatter(msgA, dstA, zeros_acc)
        aggB = sc_scatter(msgB, dstB, zeros_acc)
        return _gru(aggA, aggB, hidden, wih, bih, whh, bhh, bconv), None

    x, _ = lax.scan(step, x0, None, length=STEPS)

    out = _dec(x, W_d1, b_d1.reshape(1, H), W_d2, b_d2.reshape(1, D_OUT))
    return out[:N]
